# baseline (speedup 1.0000x reference)
import functools

import jax
import jax.numpy as jnp
from jax import lax
from jax.experimental import pallas as pl
from jax.experimental.pallas import tpu as pltpu
from jax.experimental.pallas import tpu_sc as plsc

_BATCH = 4096
_ROW = 200 * 64
_NB = 4
_LANES = 16
_UNROLL = 8


def _make_sc_kernel(n_workers, rows_per_worker):
    mesh = plsc.VectorSubcoreMesh(core_axis_name="c", subcore_axis_name="s")

    @functools.partial(
        pl.kernel,
        mesh=mesh,
        out_type=jax.ShapeDtypeStruct((_BATCH, _ROW), jnp.float32),
        scratch_types=[
            pltpu.VMEM((_ROW,), jnp.float32),
            pltpu.VMEM((_NB, _ROW), jnp.float32),
            pltpu.SemaphoreType.DMA((_NB,)),
            pltpu.SemaphoreType.DMA((_NB,)),
        ],
    )
    def sc_add(x_hbm, pos_hbm, out_hbm, pos_v, buf, in_sems, out_sems):
        wid = lax.axis_index("s") * 2 + lax.axis_index("c")
        base = wid * rows_per_worker
        n_chunks = rows_per_worker
        n_outer = n_chunks // _NB

        pltpu.sync_copy(pos_hbm, pos_v)

        for b in range(_NB):
            pltpu.async_copy(x_hbm.at[base + b], buf.at[b], in_sems.at[b])

        def outer(i, carry):
            for b in range(_NB):
                k = i * _NB + b
                row = base + k
                pltpu.make_async_copy(
                    x_hbm.at[row], buf.at[b], in_sems.at[b]
                ).wait()

                rowref = buf.at[b]

                def inner(j, c):
                    for u in range(_UNROLL):
                        sl = pl.ds((j * _UNROLL + u) * _LANES, _LANES)
                        rowref[sl] = rowref[sl] + pos_v[sl]
                    return c

                lax.fori_loop(0, _ROW // (_LANES * _UNROLL), inner, 0)

                pltpu.async_copy(buf.at[b], out_hbm.at[row], out_sems.at[b])

                @pl.when(i < n_outer - 1)
                def _prefetch():
                    pltpu.make_async_copy(
                        buf.at[b], out_hbm.at[row], out_sems.at[b]
                    ).wait()
                    pltpu.async_copy(
                        x_hbm.at[row + _NB], buf.at[b], in_sems.at[b]
                    )

            return carry

        lax.fori_loop(0, n_outer, outer, 0)

        for b in range(_NB):
            row = base + (n_outer - 1) * _NB + b
            pltpu.make_async_copy(
                buf.at[b], out_hbm.at[row], out_sems.at[b]
            ).wait()

    return sc_add


def kernel(x, pos_table):
    B, S, D = x.shape
    info = plsc.get_sparse_core_info()
    n_workers = info.num_cores * info.num_subcores
    sc_add = _make_sc_kernel(n_workers, B // n_workers)
    x2 = x.reshape(B, S * D)
    pos = pos_table[:S].reshape(S * D)
    out = sc_add(x2, pos)
    return out.reshape(B, S, D)


# speedup vs baseline: 1.5421x; 1.5421x over previous
import functools

import jax
import jax.numpy as jnp
from jax import lax
from jax.experimental import pallas as pl
from jax.experimental.pallas import tpu as pltpu
from jax.experimental.pallas import tpu_sc as plsc

_BATCH = 4096
_ROW = 200 * 64
_NB = 4
_LANES = 16
_UNROLL = 8


def _make_sc_kernel(n_workers, rows_per_worker):
    mesh = plsc.VectorSubcoreMesh(core_axis_name="c", subcore_axis_name="s")

    @functools.partial(
        pl.kernel,
        mesh=mesh,
        out_type=jax.ShapeDtypeStruct((_BATCH, _ROW), jnp.float32),
        scratch_types=[
            pltpu.VMEM((_ROW,), jnp.float32),
            pltpu.VMEM((_NB, _ROW), jnp.float32),
            pltpu.SemaphoreType.DMA((_NB,)),
            pltpu.SemaphoreType.DMA((_NB,)),
        ],
    )
    def sc_add(x_hbm, pos_hbm, out_hbm, pos_v, buf, in_sems, out_sems):
        wid = lax.axis_index("s") * 2 + lax.axis_index("c")
        base = wid * rows_per_worker
        n_chunks = rows_per_worker
        n_outer = n_chunks // _NB

        pltpu.sync_copy(pos_hbm, pos_v)

        for b in range(_NB):
            pltpu.async_copy(x_hbm.at[base + b], buf.at[b], in_sems.at[b])

        def outer(i, carry):
            for b in range(_NB):
                k = i * _NB + b
                row = base + k
                pltpu.make_async_copy(
                    x_hbm.at[row], buf.at[b], in_sems.at[b]
                ).wait()

                rowref = buf.at[b]

                @plsc.parallel_loop(0, _ROW, step=_LANES, unroll=_UNROLL)
                def _add(j):
                    sl = pl.ds(j, _LANES)
                    rowref[sl] = rowref[sl] + pos_v[sl]

                pltpu.async_copy(buf.at[b], out_hbm.at[row], out_sems.at[b])

                @pl.when(i < n_outer - 1)
                def _prefetch():
                    pltpu.make_async_copy(
                        buf.at[b], out_hbm.at[row], out_sems.at[b]
                    ).wait()
                    pltpu.async_copy(
                        x_hbm.at[row + _NB], buf.at[b], in_sems.at[b]
                    )

            return carry

        lax.fori_loop(0, n_outer, outer, 0)

        for b in range(_NB):
            row = base + (n_outer - 1) * _NB + b
            pltpu.make_async_copy(
                buf.at[b], out_hbm.at[row], out_sems.at[b]
            ).wait()

    return sc_add


def kernel(x, pos_table):
    B, S, D = x.shape
    info = plsc.get_sparse_core_info()
    n_workers = info.num_cores * info.num_subcores
    sc_add = _make_sc_kernel(n_workers, B // n_workers)
    x2 = x.reshape(B, S * D)
    pos = pos_table[:S].reshape(S * D)
    out = sc_add(x2, pos)
    return out.reshape(B, S, D)


# speedup vs baseline: 1.9892x; 1.2899x over previous
import functools

import jax
import jax.numpy as jnp
from jax import lax
from jax.experimental import pallas as pl
from jax.experimental.pallas import tpu as pltpu
from jax.experimental.pallas import tpu_sc as plsc

_BATCH = 4096
_ROW = 200 * 64
_NB = 4
_LANES = 16
_UNROLL = 8


def _make_sc_kernel(n_workers, rows_per_worker):
    mesh = plsc.VectorSubcoreMesh(core_axis_name="c", subcore_axis_name="s")

    @functools.partial(
        pl.kernel,
        mesh=mesh,
        out_type=jax.ShapeDtypeStruct((_BATCH, _ROW), jnp.float32),
        scratch_types=[
            pltpu.VMEM((_ROW,), jnp.float32),
            pltpu.VMEM((_NB, _ROW), jnp.float32),
            pltpu.SemaphoreType.DMA((_NB,)),
            pltpu.SemaphoreType.DMA((_NB,)),
        ],
    )
    def sc_add(x_hbm, pos_hbm, out_hbm, pos_v, buf, in_sems, out_sems):
        wid = lax.axis_index("s") * 2 + lax.axis_index("c")
        base = wid * rows_per_worker
        n_chunks = rows_per_worker
        n_outer = n_chunks // _NB

        pltpu.sync_copy(pos_hbm, pos_v)

        for b in range(_NB):
            pltpu.async_copy(x_hbm.at[base + b], buf.at[b], in_sems.at[b])

        def outer(i, carry):
            for b in range(_NB):
                k = i * _NB + b
                row = base + k
                pltpu.make_async_copy(
                    x_hbm.at[row], buf.at[b], in_sems.at[b]
                ).wait()

                rowref = buf.at[b]

                @plsc.parallel_loop(0, _ROW, step=_LANES, unroll=_UNROLL)
                def _add(j):
                    sl = pl.ds(j, _LANES)
                    rowref[sl] = rowref[sl] + pos_v[sl]

                pltpu.async_copy(buf.at[b], out_hbm.at[row], out_sems.at[b])

                @pl.when(i < n_outer - 1)
                def _prefetch():
                    pltpu.make_async_copy(
                        buf.at[b], out_hbm.at[row], out_sems.at[b]
                    ).wait()
                    pltpu.async_copy(
                        x_hbm.at[row + _NB], buf.at[b], in_sems.at[b]
                    )

            return carry

        lax.fori_loop(0, n_outer, outer, 0)

        for b in range(_NB):
            row = base + (n_outer - 1) * _NB + b
            pltpu.make_async_copy(
                buf.at[b], out_hbm.at[row], out_sems.at[b]
            ).wait()

    return sc_add


_B = 4096
_N = 200 * 64
_CH = 128
_Q = 4
_NCH = _B // _CH
_CHQ = _CH // _Q


def _body(pos_ref, x_hbm, out_hbm, buf, obuf, in_sems, out_sems):
    i = pl.program_id(0)
    slot = jax.lax.rem(i, 2)
    nslot = jax.lax.rem(i + 1, 2)

    def start_in(chunk, s):
        for q in range(_Q):
            pltpu.make_async_copy(
                x_hbm.at[pl.ds(chunk * _CH + q * _CHQ, _CHQ)],
                buf.at[s, pl.ds(q * _CHQ, _CHQ)],
                in_sems.at[s, q],
            ).start()

    def wait_in(chunk, s):
        for q in range(_Q):
            pltpu.make_async_copy(
                x_hbm.at[pl.ds(chunk * _CH + q * _CHQ, _CHQ)],
                buf.at[s, pl.ds(q * _CHQ, _CHQ)],
                in_sems.at[s, q],
            ).wait()

    def start_out(chunk, s):
        for q in range(_Q):
            pltpu.make_async_copy(
                obuf.at[s, pl.ds(q * _CHQ, _CHQ)],
                out_hbm.at[pl.ds(chunk * _CH + q * _CHQ, _CHQ)],
                out_sems.at[s, q],
            ).start()

    def wait_out(chunk, s):
        for q in range(_Q):
            pltpu.make_async_copy(
                obuf.at[s, pl.ds(q * _CHQ, _CHQ)],
                out_hbm.at[pl.ds(chunk * _CH + q * _CHQ, _CHQ)],
                out_sems.at[s, q],
            ).wait()

    @pl.when(i == 0)
    def _prologue():
        start_in(0, 0)

    @pl.when(i + 1 < _NCH)
    def _prefetch():
        start_in(i + 1, nslot)

    wait_in(i, slot)

    @pl.when(i >= 2)
    def _free_out():
        wait_out(i - 2, slot)

    obuf[slot] = buf[slot] + pos_ref[...]

    start_out(i, slot)

    @pl.when(i == _NCH - 1)
    def _drain():
        wait_out(i - 1, nslot)
        wait_out(i, slot)


def kernel(x, pos_table):
    B, S, D = x.shape
    x2 = x.reshape(B, S * D)
    pos = pos_table[:S].reshape(1, S * D)
    out = pl.pallas_call(
        _body,
        grid=(_NCH,),
        in_specs=[
            pl.BlockSpec((1, _N), lambda i: (0, 0)),
            pl.BlockSpec(memory_space=pl.ANY),
        ],
        out_specs=pl.BlockSpec(memory_space=pl.ANY),
        out_shape=jax.ShapeDtypeStruct((B, _N), jnp.float32),
        scratch_shapes=[
            pltpu.VMEM((2, _CH, _N), jnp.float32),
            pltpu.VMEM((2, _CH, _N), jnp.float32),
            pltpu.SemaphoreType.DMA((2, _Q)),
            pltpu.SemaphoreType.DMA((2, _Q)),
        ],
        compiler_params=pltpu.CompilerParams(
            dimension_semantics=("arbitrary",),
        ),
    )(pos, x2)
    return out.reshape(B, S, D)


# speedup vs baseline: 6.9914x; 3.5146x over previous
import jax
import jax.numpy as jnp
from jax.experimental import pallas as pl
from jax.experimental.pallas import tpu as pltpu

_B = 4096
_SD = 200 * 64
_BLK = 512


def _add_body(x_ref, pos_ref, out_ref):
    out_ref[...] = x_ref[...] + pos_ref[...]


def kernel(x, pos_table):
    B, S, D = x.shape
    xt = x.transpose(1, 2, 0).reshape(S * D, B)
    post = pos_table[:S].reshape(S * D, 1)
    out_t = pl.pallas_call(
        _add_body,
        grid=(S * D // _BLK,),
        in_specs=[
            pl.BlockSpec((_BLK, B), lambda i: (i, 0)),
            pl.BlockSpec((_BLK, 1), lambda i: (i, 0)),
        ],
        out_specs=pl.BlockSpec((_BLK, B), lambda i: (i, 0)),
        out_shape=jax.ShapeDtypeStruct((S * D, B), x.dtype),
    )(xt, post)
    return out_t.reshape(S, D, B).transpose(2, 0, 1)
